# baseline (speedup 1.0000x reference)
import jax
import jax.numpy as jnp
from jax.experimental import pallas as pl


def _onehot_body(x_ref, lo_ref, hi_ref, o_ref):
    x = x_ref[...]
    lo = lo_ref[...]
    hi = hi_ref[...]
    b = x.shape[0]
    xa = jnp.broadcast_to(x[:, 0:1], (b, 64))
    xb = jnp.broadcast_to(x[:, 1:2], (b, 64))
    xab = jnp.concatenate([xa, xb], axis=1)
    hit = jnp.logical_and(xab > lo, xab <= hi)
    o_ref[...] = hit.astype(jnp.float32)


def kernel(data, v_bins):
    n_bins = v_bins.shape[0] - 1
    n = data.size
    x = data.reshape(n // 2, 2)

    mid = v_bins[1:n_bins]
    lo1 = jnp.concatenate([jnp.full((1,), -jnp.inf, v_bins.dtype), mid])
    hi1 = jnp.concatenate([mid, jnp.full((1,), jnp.inf, v_bins.dtype)])
    lo = jnp.tile(lo1, 2).reshape(1, 2 * n_bins)
    hi = jnp.tile(hi1, 2).reshape(1, 2 * n_bins)

    block = 8192
    grid = (n // 2) // block
    out = pl.pallas_call(
        _onehot_body,
        grid=(grid,),
        in_specs=[
            pl.BlockSpec((block, 2), lambda i: (i, 0)),
            pl.BlockSpec((1, 2 * n_bins), lambda i: (0, 0)),
            pl.BlockSpec((1, 2 * n_bins), lambda i: (0, 0)),
        ],
        out_specs=pl.BlockSpec((block, 2 * n_bins), lambda i: (i, 0)),
        out_shape=jax.ShapeDtypeStruct((n // 2, 2 * n_bins), jnp.float32),
    )(x, lo, hi)
    return out.reshape(data.shape + (n_bins,))
